# Initial kernel scaffold; baseline (speedup 1.0000x reference)
import jax
import jax.numpy as jnp
from jax.experimental import pallas as pl


def kernel(R, offsets, idx_i, idx_j):
    raise NotImplementedError("write your pallas kernel here")


# speedup vs baseline: 4.0667x; 4.0667x over previous
import jax
import jax.numpy as jnp
from jax import lax
from jax.experimental import pallas as pl
from jax.experimental.pallas import tpu as pltpu
from jax.experimental.pallas import tpu_sc as plsc

N_NODES = 100000
N_EDGES = 6400000

NC = 2
NS = 16
L = 16
NW = NC * NS
D = 8

E_PER_W = N_EDGES // NW
B = 2000
NCHUNKS = E_PER_W // B
MACRO = B * 3 // (3 * L)


def _body(R_hbm, off_hbm, idx_i_hbm, idx_j_hbm, out_hbm,
          shared_R, idx_i_v, idx_j_v, off_v, posi_v, posj_v, sem):
  cid = lax.axis_index("c")
  sid = lax.axis_index("s")
  wid = sid * NC + cid

  @pl.when(sid == 0)
  def _():
    pltpu.sync_copy(R_hbm, shared_R)
  plsc.subcore_barrier()

  lanes = lax.iota(jnp.int32, L)
  rowp = []
  colp = []
  for j in range(3):
    flat = lanes + (16 * j)
    r = (flat * 21846) >> 16
    rowp.append(r)
    colp.append(flat - r * 3)

  def chunk_body(t, carry):
    base = wid * E_PER_W + t * B
    pltpu.sync_copy(idx_i_hbm.at[pl.ds(base, B)], idx_i_v)
    pltpu.sync_copy(idx_j_hbm.at[pl.ds(base, B)], idx_j_v)
    pltpu.sync_copy(off_hbm.at[pl.ds(base * 3, B * 3)], off_v)
    pltpu.async_copy(shared_R.at[idx_i_v], posi_v, sem).wait()
    pltpu.async_copy(shared_R.at[idx_j_v], posj_v, sem).wait()

    def macro_body(k, c2):
      row_base = k * L
      for j in range(3):
        rows = rowp[j] + row_base
        pj = plsc.load_gather(posj_v, [rows, colp[j]])
        pi = plsc.load_gather(posi_v, [rows, colp[j]])
        w = k * (3 * L) + j * L
        off_v[pl.ds(w, L)] = off_v[pl.ds(w, L)] + (pj - pi)
      return c2

    lax.fori_loop(0, MACRO, macro_body, 0, unroll=2)
    pltpu.sync_copy(off_v, out_hbm.at[pl.ds(base * 3, B * 3)])
    return carry

  lax.fori_loop(0, NCHUNKS, chunk_body, 0)


@jax.jit
def kernel(R, offsets, idx_i, idx_j):
  mesh = plsc.VectorSubcoreMesh(core_axis_name="c", subcore_axis_name="s",
                                num_cores=NC, num_subcores=NS)
  R_pad = jnp.pad(R, ((0, 0), (0, D - 3)))
  out_flat = pl.kernel(
      _body,
      out_type=jax.ShapeDtypeStruct((N_EDGES * 3,), jnp.float32),
      mesh=mesh,
      compiler_params=pltpu.CompilerParams(use_tc_tiling_on_sc=False,
                                           needs_layout_passes=False),
      scratch_types=[
          pltpu.VMEM_SHARED((N_NODES, D), jnp.float32),
          pltpu.VMEM((B,), jnp.int32),
          pltpu.VMEM((B,), jnp.int32),
          pltpu.VMEM((B * 3,), jnp.float32),
          pltpu.VMEM((B, D), jnp.float32),
          pltpu.VMEM((B, D), jnp.float32),
          pltpu.SemaphoreType.DMA,
      ],
  )(R_pad, offsets.reshape(-1), idx_i, idx_j)
  return out_flat.reshape(N_EDGES, 3)
